# baseline (speedup 1.0000x reference)
import jax
import jax.numpy as jnp
from jax.experimental import pallas as pl
from jax.experimental.pallas import tpu as pltpu

PRE_SEQ_LEN = 128
HIDDEN = 1024
NUM_LAYERS = 24
OUT_DIM = NUM_LAYERS * 2 * HIDDEN
BATCH = 16
BLOCK_N = 2048
NB = OUT_DIM // BLOCK_N


_WIDTHS = [512, 512, 1024] + [2048] * 22 + [1024, 512, 512]
_OFFS = [sum(_WIDTHS[:i]) for i in range(len(_WIDTHS))]
_NBLK = len(_WIDTHS)


def _fused_kernel(prefix_ref, emb_ref, w1_ref, b1_ref, w2_hbm, b2_ref,
                  out_hbm, htab_ref, onehot_ref,
                  w2buf0, w2buf1, obuf0, obuf1,
                  rsem0, rsem1, wsem0, wsem1):
    w2bufs = (w2buf0, w2buf1)
    obufs = (obuf0, obuf1)
    rsems = (rsem0, rsem1)
    wsems = (wsem0, wsem1)

    def rd(j):
        return pltpu.make_async_copy(
            w2_hbm.at[:, pl.ds(_OFFS[j], _WIDTHS[j])],
            w2bufs[j % 2].at[:, pl.ds(0, _WIDTHS[j])],
            rsems[j % 2])

    def wr(j):
        return pltpu.make_async_copy(
            obufs[j % 2].at[:, :, pl.ds(0, _WIDTHS[j])],
            out_hbm.at[:, :, pl.ds(_OFFS[j], _WIDTHS[j])],
            wsems[j % 2])

    rd(0).start()
    rd(1).start()

    h = jnp.dot(emb_ref[...], w1_ref[...], preferred_element_type=jnp.float32)
    htab_ref[...] = jnp.tanh(h + b1_ref[...]).astype(jnp.bfloat16)
    ids = prefix_ref[...]
    iota = jax.lax.broadcasted_iota(
        jnp.int32, (BATCH, PRE_SEQ_LEN, PRE_SEQ_LEN), 2)
    onehot_ref[...] = (ids[:, :, None] == iota).astype(jnp.bfloat16)

    for j in range(_NBLK):
        rd(j).wait()
        if j >= 2:
            wr(j - 2).wait()
        w = _WIDTHS[j]
        t = jnp.dot(htab_ref[...],
                    w2bufs[j % 2][:, pl.ds(0, w)].astype(jnp.bfloat16),
                    preferred_element_type=jnp.float32).astype(jnp.bfloat16)
        obufs[j % 2][:, :, pl.ds(0, w)] = jax.lax.dot_general(
            onehot_ref[...], t,
            dimension_numbers=(((2,), (0,)), ((), ())),
            preferred_element_type=jnp.float32
        ) + b2_ref[:, pl.ds(_OFFS[j], w)]
        wr(j).start()
        if j + 2 < _NBLK:
            rd(j + 2).start()
    wr(_NBLK - 2).wait()
    wr(_NBLK - 1).wait()


@jax.jit
def kernel(prefix, emb, W1, b1, W2, b2):
    prefix = prefix.astype(jnp.int32)
    b1r = b1.reshape(1, HIDDEN)
    b2r = b2.reshape(1, OUT_DIM)
    out = pl.pallas_call(
        _fused_kernel,
        in_specs=[
            pl.BlockSpec(memory_space=pltpu.VMEM),
            pl.BlockSpec(memory_space=pltpu.VMEM),
            pl.BlockSpec(memory_space=pltpu.VMEM),
            pl.BlockSpec(memory_space=pltpu.VMEM),
            pl.BlockSpec(memory_space=pl.ANY),
            pl.BlockSpec(memory_space=pltpu.VMEM),
        ],
        out_specs=pl.BlockSpec(memory_space=pl.ANY),
        out_shape=jax.ShapeDtypeStruct((BATCH, PRE_SEQ_LEN, OUT_DIM),
                                       jnp.float32),
        scratch_shapes=[
            pltpu.VMEM((PRE_SEQ_LEN, HIDDEN), jnp.bfloat16),
            pltpu.VMEM((BATCH, PRE_SEQ_LEN, PRE_SEQ_LEN), jnp.bfloat16),
            pltpu.VMEM((HIDDEN, BLOCK_N), jnp.float32),
            pltpu.VMEM((HIDDEN, BLOCK_N), jnp.float32),
            pltpu.VMEM((BATCH, PRE_SEQ_LEN, BLOCK_N), jnp.float32),
            pltpu.VMEM((BATCH, PRE_SEQ_LEN, BLOCK_N), jnp.float32),
            pltpu.SemaphoreType.DMA,
            pltpu.SemaphoreType.DMA,
            pltpu.SemaphoreType.DMA,
            pltpu.SemaphoreType.DMA,
        ],
    )(prefix, emb, W1, b1r, W2, b2r)
    return out


# speedup vs baseline: 1.0109x; 1.0109x over previous
import jax
import jax.numpy as jnp
from jax.experimental import pallas as pl
from jax.experimental.pallas import tpu as pltpu

PRE_SEQ_LEN = 128
HIDDEN = 1024
NUM_LAYERS = 24
OUT_DIM = NUM_LAYERS * 2 * HIDDEN
BATCH = 16
BLOCK_N = 2048
NB = OUT_DIM // BLOCK_N


_WIDTHS = [BLOCK_N] * NB
_OFFS = [sum(_WIDTHS[:i]) for i in range(len(_WIDTHS))]
_NBLK = len(_WIDTHS)


def _fused_kernel(prefix_ref, emb_ref, w1_ref, b1_ref, w2_hbm, b2_ref,
                  out_hbm, htab_ref, onehot_ref,
                  w2buf0, w2buf1, obuf0, obuf1,
                  rsem0, rsem1, wsem0, wsem1):
    w2bufs = (w2buf0, w2buf1)
    obufs = (obuf0, obuf1)
    rsems = (rsem0, rsem1)
    wsems = (wsem0, wsem1)

    def rd(j):
        return pltpu.make_async_copy(
            w2_hbm.at[:, pl.ds(_OFFS[j], _WIDTHS[j])],
            w2bufs[j % 2].at[:, pl.ds(0, _WIDTHS[j])],
            rsems[j % 2])

    def wr(j):
        return pltpu.make_async_copy(
            obufs[j % 2].at[:, :, pl.ds(0, _WIDTHS[j])],
            out_hbm.at[:, :, pl.ds(_OFFS[j], _WIDTHS[j])],
            wsems[j % 2])

    rd(0).start()
    rd(1).start()

    h = jnp.dot(emb_ref[...], w1_ref[...], preferred_element_type=jnp.float32)
    htab_ref[...] = jnp.tanh(h + b1_ref[...]).astype(jnp.bfloat16)
    ids = prefix_ref[...]
    iota = jax.lax.broadcasted_iota(
        jnp.int32, (BATCH, PRE_SEQ_LEN, PRE_SEQ_LEN), 2)
    onehot_ref[...] = (ids[:, :, None] == iota).astype(jnp.bfloat16)

    for j in range(_NBLK):
        rd(j).wait()
        if j >= 2:
            wr(j - 2).wait()
        w = _WIDTHS[j]
        t = jnp.dot(htab_ref[...],
                    w2bufs[j % 2][:, pl.ds(0, w)].astype(jnp.bfloat16),
                    preferred_element_type=jnp.float32).astype(jnp.bfloat16)
        obufs[j % 2][:, :, pl.ds(0, w)] = jax.lax.dot_general(
            onehot_ref[...], t,
            dimension_numbers=(((2,), (0,)), ((), ())),
            preferred_element_type=jnp.float32
        ) + b2_ref[:, pl.ds(_OFFS[j], w)]
        wr(j).start()
        if j + 2 < _NBLK:
            rd(j + 2).start()
    wr(_NBLK - 2).wait()
    wr(_NBLK - 1).wait()


@jax.jit
def kernel(prefix, emb, W1, b1, W2, b2):
    prefix = prefix.astype(jnp.int32)
    b1r = b1.reshape(1, HIDDEN)
    b2r = b2.reshape(1, OUT_DIM)
    out = pl.pallas_call(
        _fused_kernel,
        in_specs=[
            pl.BlockSpec(memory_space=pltpu.VMEM),
            pl.BlockSpec(memory_space=pltpu.VMEM),
            pl.BlockSpec(memory_space=pltpu.VMEM),
            pl.BlockSpec(memory_space=pltpu.VMEM),
            pl.BlockSpec(memory_space=pl.ANY),
            pl.BlockSpec(memory_space=pltpu.VMEM),
        ],
        out_specs=pl.BlockSpec(memory_space=pl.ANY),
        out_shape=jax.ShapeDtypeStruct((BATCH, PRE_SEQ_LEN, OUT_DIM),
                                       jnp.float32),
        scratch_shapes=[
            pltpu.VMEM((PRE_SEQ_LEN, HIDDEN), jnp.bfloat16),
            pltpu.VMEM((BATCH, PRE_SEQ_LEN, PRE_SEQ_LEN), jnp.bfloat16),
            pltpu.VMEM((HIDDEN, BLOCK_N), jnp.float32),
            pltpu.VMEM((HIDDEN, BLOCK_N), jnp.float32),
            pltpu.VMEM((BATCH, PRE_SEQ_LEN, BLOCK_N), jnp.float32),
            pltpu.VMEM((BATCH, PRE_SEQ_LEN, BLOCK_N), jnp.float32),
            pltpu.SemaphoreType.DMA,
            pltpu.SemaphoreType.DMA,
            pltpu.SemaphoreType.DMA,
            pltpu.SemaphoreType.DMA,
        ],
    )(prefix, emb, W1, b1r, W2, b2r)
    return out
